# baseline (speedup 1.0000x reference)
import functools

import jax
import jax.numpy as jnp
from jax import lax
from jax.experimental import pallas as pl
from jax.experimental.pallas import tpu as pltpu


def _argmin_body(flat_ref, cb_ref, cn_ref, idx_ref, s_ref, sval_ref, sidx_ref):
    kb = pl.program_id(1)
    tm = flat_ref.shape[0]
    tk = cb_ref.shape[0]

    @pl.when(kb == 0)
    def _init():
        s_ref[...] = jnp.sum(flat_ref[...] ** 2, axis=1, keepdims=True)
        sval_ref[...] = jnp.full((tm, 1), jnp.inf, jnp.float32)
        sidx_ref[...] = jnp.zeros((tm, 1), jnp.int32)

    mm = lax.dot_general(
        flat_ref[...], cb_ref[...], (((1,), (1,)), ((), ())),
        preferred_element_type=jnp.float32)
    dist = s_ref[...] - 2.0 * mm + cn_ref[...]
    m = jnp.min(dist, axis=1, keepdims=True)
    iota = lax.broadcasted_iota(jnp.int32, (tm, tk), 1) + kb * tk
    tile_idx = jnp.min(jnp.where(dist == m, iota, jnp.int32(2**30)),
                       axis=1, keepdims=True)
    better = m < sval_ref[...]
    sval_ref[...] = jnp.where(better, m, sval_ref[...])
    sidx_ref[...] = jnp.where(better, tile_idx, sidx_ref[...])

    @pl.when(kb == pl.num_programs(1) - 1)
    def _fin():
        idx_ref[...] = sidx_ref[...]


@functools.partial(jax.jit, static_argnames=("tm", "tk"))
def _argmin_call(flat, cb, cn, tm=768, tk=1024):
    n, d = flat.shape
    k = cb.shape[0]
    grid = (n // tm, k // tk)
    out = pl.pallas_call(
        _argmin_body,
        grid=grid,
        in_specs=[
            pl.BlockSpec((tm, d), lambda tb, kb: (tb, 0)),
            pl.BlockSpec((tk, d), lambda tb, kb: (kb, 0)),
            pl.BlockSpec((1, tk), lambda tb, kb: (0, kb)),
        ],
        out_specs=pl.BlockSpec((tm, 1), lambda tb, kb: (tb, 0)),
        out_shape=jax.ShapeDtypeStruct((n, 1), jnp.int32),
        scratch_shapes=[
            pltpu.VMEM((tm, 1), jnp.float32),
            pltpu.VMEM((tm, 1), jnp.float32),
            pltpu.VMEM((tm, 1), jnp.int32),
        ],
        compiler_params=pltpu.CompilerParams(
            dimension_semantics=("arbitrary", "arbitrary")),
    )(flat, cb, cn.reshape(1, k))
    return out[:, 0]


def kernel(x, codebooks):
    b, d, t = x.shape
    num_q, k, _ = codebooks.shape
    cn = jnp.sum(codebooks ** 2, axis=-1)

    quantized_out = jnp.zeros_like(x)
    residual = x
    all_indices = []
    all_losses = []
    all_perp = []
    for q in range(num_q):
        flat = jnp.transpose(residual, (0, 2, 1)).reshape(-1, d)
        n = flat.shape[0]
        idx = _argmin_call(flat, codebooks[q], cn[q],
                           tm=768 if n % 768 == 0 else n,
                           tk=1024 if k % 1024 == 0 else k)
        x_d_flat = jnp.take(codebooks[q], idx, axis=0)
        commit_loss = jnp.mean((flat - x_d_flat) ** 2)
        onehot = jax.nn.one_hot(idx, k, dtype=jnp.float32)
        prob = jnp.mean(onehot, axis=0)
        perplexity = jnp.exp(-jnp.sum(prob * jnp.log(prob + 1e-7)))
        x_d = jnp.transpose(x_d_flat.reshape(b, t, d), (0, 2, 1))
        residual = residual - x_d
        quantized_out = quantized_out + x_d
        all_indices.append(idx.reshape(b, t))
        all_losses.append(commit_loss)
        all_perp.append(perplexity)
    all_indices = jnp.stack(all_indices, axis=-1)
    vq_loss = sum(all_losses) / len(all_losses)
    perplexity = sum(all_perp) / len(all_perp)
    return quantized_out, all_indices, vq_loss, perplexity


# speedup vs baseline: 1.1799x; 1.1799x over previous
import functools

import jax
import jax.numpy as jnp
from jax import lax
from jax.experimental import pallas as pl
from jax.experimental.pallas import tpu as pltpu
from jax.experimental.pallas import tpu_sc as plsc

NC = 2
NS = 16
NW = NC * NS
L = 16


def _argmin_body(flat_ref, cb_ref, cn_ref, idx_ref, s_ref, sval_ref, sidx_ref):
    kb = pl.program_id(1)
    tm = flat_ref.shape[0]
    tk = cb_ref.shape[0]

    @pl.when(kb == 0)
    def _init():
        s_ref[...] = jnp.sum(flat_ref[...] ** 2, axis=1, keepdims=True)
        sval_ref[...] = jnp.full((tm, 1), jnp.inf, jnp.float32)
        sidx_ref[...] = jnp.zeros((tm, 1), jnp.int32)

    mm = lax.dot_general(
        flat_ref[...], cb_ref[...], (((1,), (1,)), ((), ())),
        preferred_element_type=jnp.float32)
    dist = s_ref[...] - 2.0 * mm + cn_ref[...]
    m = jnp.min(dist, axis=1, keepdims=True)
    iota = lax.broadcasted_iota(jnp.int32, (tm, tk), 1) + kb * tk
    tile_idx = jnp.min(jnp.where(dist == m, iota, jnp.int32(2**30)),
                       axis=1, keepdims=True)
    better = m < sval_ref[...]
    sval_ref[...] = jnp.where(better, m, sval_ref[...])
    sidx_ref[...] = jnp.where(better, tile_idx, sidx_ref[...])

    @pl.when(kb == pl.num_programs(1) - 1)
    def _fin():
        idx_ref[...] = sidx_ref[...]


def _argmin_call(flat, cb, cn, tm=768, tk=1024):
    n, d = flat.shape
    k = cb.shape[0]
    grid = (n // tm, k // tk)
    out = pl.pallas_call(
        _argmin_body,
        grid=grid,
        in_specs=[
            pl.BlockSpec((tm, d), lambda tb, kb: (tb, 0)),
            pl.BlockSpec((tk, d), lambda tb, kb: (kb, 0)),
            pl.BlockSpec((1, tk), lambda tb, kb: (0, kb)),
        ],
        out_specs=pl.BlockSpec((tm, 1), lambda tb, kb: (tb, 0)),
        out_shape=jax.ShapeDtypeStruct((n, 1), jnp.int32),
        scratch_shapes=[
            pltpu.VMEM((tm, 1), jnp.float32),
            pltpu.VMEM((tm, 1), jnp.float32),
            pltpu.VMEM((tm, 1), jnp.int32),
        ],
        compiler_params=pltpu.CompilerParams(
            dimension_semantics=("arbitrary", "arbitrary")),
    )(flat, cb, cn.reshape(1, k))
    return out[:, 0]


@functools.lru_cache(maxsize=None)
def _make_sc_stage(n, d, k, final):
    perw = n // NW
    ch = 96
    nch = perw // ch
    assert perw % ch == 0 and d % L == 0 and k % L == 0

    outs = [jax.ShapeDtypeStruct((n, d), jnp.float32),
            jax.ShapeDtypeStruct((NW, k), jnp.float32),
            jax.ShapeDtypeStruct((NW, L), jnp.float32)]
    if final:
        outs.append(jax.ShapeDtypeStruct((n, d), jnp.float32))
    scratch = [pltpu.VMEM((ch,), jnp.int32),
               pltpu.VMEM((ch, d), jnp.float32),
               pltpu.VMEM((ch, d), jnp.float32),
               pltpu.VMEM((k + L,), jnp.float32),
               pltpu.VMEM((L,), jnp.float32),
               pltpu.SemaphoreType.DMA]
    if final:
        scratch.insert(3, pltpu.VMEM((ch, d), jnp.float32))

    def body(*args):
        if final:
            (cb_hbm, idx_hbm, r_hbm, x0_hbm, rout_hbm, counts_hbm, loss_hbm,
             qout_hbm, idx_v, rows_v, r_v, x_v, counts_v, acc_v, sem) = args
        else:
            (cb_hbm, idx_hbm, r_hbm, rout_hbm, counts_hbm, loss_hbm,
             idx_v, rows_v, r_v, counts_v, acc_v, sem) = args
        wid = lax.axis_index("s") * NC + lax.axis_index("c")
        base = wid * perw
        zeros = jnp.zeros((L,), jnp.float32)
        ones = jnp.ones((L,), jnp.float32)
        lane = lax.iota(jnp.int32, L)

        def zbody(i, c):
            counts_v[pl.ds(i * L, L)] = zeros
            return c
        lax.fori_loop(0, k // L + 1, zbody, 0)
        acc_v[...] = zeros

        for c in range(nch):
            cbase = base + c * ch
            pltpu.sync_copy(idx_hbm.at[pl.ds(cbase, ch)], idx_v)
            pltpu.async_copy(cb_hbm.at[idx_v], rows_v, sem).wait()
            pltpu.sync_copy(r_hbm.at[pl.ds(cbase, ch)], r_v)
            if final:
                pltpu.sync_copy(x0_hbm.at[pl.ds(cbase, ch)], x_v)

            def tbody(t, acc):
                for j in range(d // L):
                    sl = pl.ds(j * L, L)
                    dlt = r_v[t, sl] - rows_v[t, sl]
                    r_v[t, sl] = dlt
                    if final:
                        x_v[t, sl] = x_v[t, sl] - dlt
                    acc = acc + dlt * dlt
                return acc
            acc_v[...] = lax.fori_loop(0, ch, tbody, acc_v[...])

            pltpu.sync_copy(r_v, rout_hbm.at[pl.ds(cbase, ch)])
            if final:
                pltpu.sync_copy(x_v, qout_hbm.at[pl.ds(cbase, ch)])
            one0 = jnp.where(lane == 0, ones, zeros)
            for v in range(ch // L):
                iv = idx_v[pl.ds(v * L, L)]
                for j in range(L):
                    si = iv[j]
                    cv = counts_v[pl.ds(si, L)]
                    counts_v[pl.ds(si, L)] = cv + one0

        pltpu.sync_copy(counts_v.at[pl.ds(0, k)], counts_hbm.at[wid])
        pltpu.sync_copy(acc_v, loss_hbm.at[wid])

    mesh = plsc.VectorSubcoreMesh(core_axis_name="c", subcore_axis_name="s")
    return pl.kernel(body, out_type=tuple(outs), mesh=mesh,
                     scratch_types=tuple(scratch))


def kernel(x, codebooks):
    b, d, t = x.shape
    nq, k, _ = codebooks.shape
    n = b * t
    cn = jnp.sum(codebooks ** 2, axis=-1)
    flat0 = jnp.transpose(x, (0, 2, 1)).reshape(n, d)

    r = flat0
    idxs, losses, perps = [], [], []
    qout_flat = None
    for q in range(nq):
        idx = _argmin_call(r, codebooks[q], cn[q],
                           tm=768 if n % 768 == 0 else n,
                           tk=1024 if k % 1024 == 0 else k)
        final = q == nq - 1
        sc = _make_sc_stage(n, d, k, final)
        if final:
            r, counts, losspart, qout_flat = sc(codebooks[q], idx, r, flat0)
        else:
            r, counts, losspart = sc(codebooks[q], idx, r)
        idxs.append(idx.reshape(b, t))
        losses.append(jnp.sum(losspart) / (n * d))
        prob = jnp.sum(counts, axis=0) / n
        perps.append(jnp.exp(-jnp.sum(prob * jnp.log(prob + 1e-7))))

    all_indices = jnp.stack(idxs, axis=-1)
    vq_loss = sum(losses) / nq
    perplexity = sum(perps) / nq
    quantized_out = jnp.transpose(qout_flat.reshape(b, t, d), (0, 2, 1))
    return quantized_out, all_indices, vq_loss, perplexity


# speedup vs baseline: 1.6887x; 1.4312x over previous
import functools

import jax
import jax.numpy as jnp
from jax import lax
from jax.experimental import pallas as pl
from jax.experimental.pallas import tpu as pltpu
from jax.experimental.pallas import tpu_sc as plsc

NC = 2
NS = 16
NW = NC * NS
L = 16


def _argmin_body(flat_ref, cb_ref, cn_ref, idx_ref, s_ref, sval_ref, sidx_ref):
    kb = pl.program_id(1)
    tm = flat_ref.shape[0]
    tk = cb_ref.shape[0]

    @pl.when(kb == 0)
    def _init():
        s_ref[...] = jnp.sum(flat_ref[...] ** 2, axis=1, keepdims=True)
        sval_ref[...] = jnp.full((tm, 1), jnp.inf, jnp.float32)
        sidx_ref[...] = jnp.zeros((tm, 1), jnp.float32)

    mm = lax.dot_general(
        flat_ref[...], cb_ref[...], (((1,), (1,)), ((), ())),
        preferred_element_type=jnp.float32)
    dist = s_ref[...] - 2.0 * mm + cn_ref[...]
    m = jnp.min(dist, axis=1, keepdims=True)
    iota = lax.broadcasted_iota(jnp.int32, (tm, tk), 1).astype(jnp.float32)
    tile_idx = (jnp.min(jnp.where(dist == m, iota, jnp.float32(2**30)),
                        axis=1, keepdims=True)
                + jnp.float32(kb * tk))
    better = m < sval_ref[...]
    sval_ref[...] = jnp.where(better, m, sval_ref[...])
    sidx_ref[...] = jnp.where(better, tile_idx, sidx_ref[...])

    @pl.when(kb == pl.num_programs(1) - 1)
    def _fin():
        idx_ref[...] = sidx_ref[...].astype(jnp.int32)


def _argmin_call(flat, cb, cn, tm=4608, tk=2048):
    n, d = flat.shape
    k = cb.shape[0]
    grid = (n // tm, k // tk)
    out = pl.pallas_call(
        _argmin_body,
        grid=grid,
        in_specs=[
            pl.BlockSpec((tm, d), lambda tb, kb: (tb, 0)),
            pl.BlockSpec((tk, d), lambda tb, kb: (kb, 0)),
            pl.BlockSpec((1, tk), lambda tb, kb: (0, kb)),
        ],
        out_specs=pl.BlockSpec((tm, 1), lambda tb, kb: (tb, 0)),
        out_shape=jax.ShapeDtypeStruct((n, 1), jnp.int32),
        scratch_shapes=[
            pltpu.VMEM((tm, 1), jnp.float32),
            pltpu.VMEM((tm, 1), jnp.float32),
            pltpu.VMEM((tm, 1), jnp.float32),
        ],
        compiler_params=pltpu.CompilerParams(
            dimension_semantics=("arbitrary", "arbitrary")),
    )(flat, cb, cn.reshape(1, k))
    return out[:, 0]


@functools.lru_cache(maxsize=None)
def _make_sc_stage(n, d, k, final):
    perw = n // NW
    ch = 96
    nch = perw // ch
    assert perw % ch == 0 and d % L == 0 and k % L == 0

    outs = [jax.ShapeDtypeStruct((n, d), jnp.float32),
            jax.ShapeDtypeStruct((NW, k), jnp.float32),
            jax.ShapeDtypeStruct((NW, L), jnp.float32)]
    if final:
        outs.append(jax.ShapeDtypeStruct((n, d), jnp.float32))
    scratch = [pltpu.VMEM((ch,), jnp.int32),
               pltpu.VMEM((ch, d), jnp.float32),
               pltpu.VMEM((ch, d), jnp.float32),
               pltpu.VMEM((k + L,), jnp.float32),
               pltpu.VMEM((L,), jnp.float32),
               pltpu.SemaphoreType.DMA]
    if final:
        scratch.insert(3, pltpu.VMEM((ch, d), jnp.float32))

    def body(*args):
        if final:
            (cb_hbm, idx_hbm, r_hbm, x0_hbm, rout_hbm, counts_hbm, loss_hbm,
             qout_hbm, idx_v, rows_v, r_v, x_v, counts_v, acc_v, sem) = args
        else:
            (cb_hbm, idx_hbm, r_hbm, rout_hbm, counts_hbm, loss_hbm,
             idx_v, rows_v, r_v, counts_v, acc_v, sem) = args
        wid = lax.axis_index("s") * NC + lax.axis_index("c")
        base = wid * perw
        zeros = jnp.zeros((L,), jnp.float32)
        ones = jnp.ones((L,), jnp.float32)
        lane = lax.iota(jnp.int32, L)

        def zbody(i, c):
            counts_v[pl.ds(i * L, L)] = zeros
            return c
        lax.fori_loop(0, k // L + 1, zbody, 0)
        acc_v[...] = zeros

        for c in range(nch):
            cbase = base + c * ch
            pltpu.sync_copy(idx_hbm.at[pl.ds(cbase, ch)], idx_v)
            pltpu.async_copy(cb_hbm.at[idx_v], rows_v, sem).wait()
            pltpu.sync_copy(r_hbm.at[pl.ds(cbase, ch)], r_v)
            if final:
                pltpu.sync_copy(x0_hbm.at[pl.ds(cbase, ch)], x_v)

            def tbody(t, acc):
                for j in range(d // L):
                    sl = pl.ds(j * L, L)
                    dlt = r_v[t, sl] - rows_v[t, sl]
                    r_v[t, sl] = dlt
                    if final:
                        x_v[t, sl] = x_v[t, sl] - dlt
                    acc = acc + dlt * dlt
                return acc
            acc_v[...] = lax.fori_loop(0, ch, tbody, acc_v[...])

            pltpu.sync_copy(r_v, rout_hbm.at[pl.ds(cbase, ch)])
            if final:
                pltpu.sync_copy(x_v, qout_hbm.at[pl.ds(cbase, ch)])
            one0 = jnp.where(lane == 0, ones, zeros)
            for v in range(ch // L):
                iv = idx_v[pl.ds(v * L, L)]
                for j in range(L):
                    si = iv[j]
                    cv = counts_v[pl.ds(si, L)]
                    counts_v[pl.ds(si, L)] = cv + one0

        pltpu.sync_copy(counts_v.at[pl.ds(0, k)], counts_hbm.at[wid])
        pltpu.sync_copy(acc_v, loss_hbm.at[wid])

    mesh = plsc.VectorSubcoreMesh(core_axis_name="c", subcore_axis_name="s")
    return pl.kernel(body, out_type=tuple(outs), mesh=mesh,
                     scratch_types=tuple(scratch))


def kernel(x, codebooks):
    b, d, t = x.shape
    nq, k, _ = codebooks.shape
    n = b * t
    cn = jnp.sum(codebooks ** 2, axis=-1)
    flat0 = jnp.transpose(x, (0, 2, 1)).reshape(n, d)

    r = flat0
    idxs, losses, perps = [], [], []
    qout_flat = None
    for q in range(nq):
        idx = _argmin_call(r, codebooks[q], cn[q],
                           tm=4608 if n % 4608 == 0 else n,
                           tk=2048 if k % 2048 == 0 else k)
        final = q == nq - 1
        sc = _make_sc_stage(n, d, k, final)
        if final:
            r, counts, losspart, qout_flat = sc(codebooks[q], idx, r, flat0)
        else:
            r, counts, losspart = sc(codebooks[q], idx, r)
        idxs.append(idx.reshape(b, t))
        losses.append(jnp.sum(losspart) / (n * d))
        prob = jnp.sum(counts, axis=0) / n
        perps.append(jnp.exp(-jnp.sum(prob * jnp.log(prob + 1e-7))))

    all_indices = jnp.stack(idxs, axis=-1)
    vq_loss = sum(losses) / nq
    perplexity = sum(perps) / nq
    quantized_out = jnp.transpose(qout_flat.reshape(b, t, d), (0, 2, 1))
    return quantized_out, all_indices, vq_loss, perplexity


# speedup vs baseline: 1.8097x; 1.0717x over previous
import functools

import jax
import jax.numpy as jnp
from jax import lax
from jax.experimental import pallas as pl
from jax.experimental.pallas import tpu as pltpu
from jax.experimental.pallas import tpu_sc as plsc

NC = 2
NS = 16
NW = NC * NS
L = 16


def _make_argmin_body(has_xd):
    def body(*refs):
        if has_xd:
            (flat_ref, xd_ref, cb_ref, cn_ref,
             idx_ref, flat_out, s_out, f_ref, s_ref, sval_ref, sidx_ref) = refs
        else:
            (flat_ref, cb_ref, cn_ref,
             idx_ref, f_ref, s_ref, sval_ref, sidx_ref) = refs
        kb = pl.program_id(1)
        tm = f_ref.shape[0]
        tk = cb_ref.shape[0]

        @pl.when(kb == 0)
        def _init():
            if has_xd:
                flat = flat_ref[...] - xd_ref[...]
                f_ref[...] = flat
                flat_out[...] = flat
            else:
                flat = flat_ref[...]
                f_ref[...] = flat
            s = jnp.sum(flat ** 2, axis=1, keepdims=True)
            s_ref[...] = s
            if has_xd:
                s_out[...] = s
            sval_ref[...] = jnp.full((tm, 1), jnp.inf, jnp.float32)
            sidx_ref[...] = jnp.zeros((tm, 1), jnp.float32)

        mm = lax.dot_general(
            f_ref[...], cb_ref[...], (((1,), (1,)), ((), ())),
            preferred_element_type=jnp.float32)
        dist = s_ref[...] - 2.0 * mm + cn_ref[...]
        m = jnp.min(dist, axis=1, keepdims=True)
        iota = lax.broadcasted_iota(jnp.int32, (tm, tk), 1).astype(jnp.float32)
        tile_idx = (jnp.min(jnp.where(dist == m, iota, jnp.float32(2**30)),
                            axis=1, keepdims=True)
                    + jnp.float32(kb * tk))
        better = m < sval_ref[...]
        sval_ref[...] = jnp.where(better, m, sval_ref[...])
        sidx_ref[...] = jnp.where(better, tile_idx, sidx_ref[...])

        @pl.when(kb == pl.num_programs(1) - 1)
        def _fin():
            idx_ref[...] = sidx_ref[...].astype(jnp.int32)

    return body


def _argmin_call(flat, xd, cb, cn, tm, tk):
    n, d = flat.shape
    k = cb.shape[0]
    grid = (n // tm, k // tk)
    has_xd = xd is not None
    in_specs = [pl.BlockSpec((tm, d), lambda tb, kb: (tb, 0))]
    args = [flat]
    if has_xd:
        in_specs.append(pl.BlockSpec((tm, d), lambda tb, kb: (tb, 0)))
        args.append(xd)
    in_specs += [
        pl.BlockSpec((tk, d), lambda tb, kb: (kb, 0)),
        pl.BlockSpec((1, tk), lambda tb, kb: (0, kb)),
    ]
    args += [cb, cn.reshape(1, k)]
    out_specs = [pl.BlockSpec((tm, 1), lambda tb, kb: (tb, 0))]
    out_shape = [jax.ShapeDtypeStruct((n, 1), jnp.int32)]
    if has_xd:
        out_specs += [pl.BlockSpec((tm, d), lambda tb, kb: (tb, 0)),
                      pl.BlockSpec((tm, 1), lambda tb, kb: (tb, 0))]
        out_shape += [jax.ShapeDtypeStruct((n, d), jnp.float32),
                      jax.ShapeDtypeStruct((n, 1), jnp.float32)]
    out = pl.pallas_call(
        _make_argmin_body(has_xd),
        grid=grid,
        in_specs=in_specs,
        out_specs=out_specs,
        out_shape=out_shape,
        scratch_shapes=[
            pltpu.VMEM((tm, d), jnp.float32),
            pltpu.VMEM((tm, 1), jnp.float32),
            pltpu.VMEM((tm, 1), jnp.float32),
            pltpu.VMEM((tm, 1), jnp.float32),
        ],
        compiler_params=pltpu.CompilerParams(
            dimension_semantics=("arbitrary", "arbitrary")),
    )(*args)
    return tuple(out)


@functools.lru_cache(maxsize=None)
def _make_sc_gather(n, d, k):
    perw = n // NW
    ch = 96
    nch = perw // ch
    assert perw % ch == 0

    def body(cb_hbm, idx_hbm, xd_hbm, idx_v, rows0, rows1, sem0, sem1):
        wid = lax.axis_index("s") * NC + lax.axis_index("c")
        base = wid * perw
        pltpu.sync_copy(idx_hbm.at[pl.ds(base, perw)], idx_v)
        rows = (rows0, rows1)
        sems = (sem0, sem1)
        descs = [None, None]
        for c in range(nch):
            bb = c % 2
            descs[bb] = pltpu.async_copy(
                cb_hbm.at[idx_v.at[pl.ds(c * ch, ch)]], rows[bb], sems[bb])
            if c > 0:
                descs[1 - bb].wait()
                pltpu.sync_copy(rows[1 - bb],
                                xd_hbm.at[pl.ds(base + (c - 1) * ch, ch)])
        descs[(nch - 1) % 2].wait()
        pltpu.sync_copy(rows[(nch - 1) % 2],
                        xd_hbm.at[pl.ds(base + (nch - 1) * ch, ch)])

    mesh = plsc.VectorSubcoreMesh(core_axis_name="c", subcore_axis_name="s")
    return pl.kernel(
        body,
        out_type=jax.ShapeDtypeStruct((n, d), jnp.float32),
        mesh=mesh,
        scratch_types=(pltpu.VMEM((perw,), jnp.int32),
                       pltpu.VMEM((ch, d), jnp.float32),
                       pltpu.VMEM((ch, d), jnp.float32),
                       pltpu.SemaphoreType.DMA,
                       pltpu.SemaphoreType.DMA))


@functools.lru_cache(maxsize=None)
def _make_sc_final(n, d, k):
    perw = n // NW
    ch = 96
    nch = perw // ch
    assert perw % ch == 0

    def body(cb_hbm, idx_hbm, r_hbm, x0_hbm, qout_hbm, loss_hbm,
             idx_v, rows_v, r_v, x_v, acc_v, sem):
        wid = lax.axis_index("s") * NC + lax.axis_index("c")
        base = wid * perw
        zeros = jnp.zeros((L,), jnp.float32)
        acc_v[...] = zeros
        for c in range(nch):
            cbase = base + c * ch
            pltpu.sync_copy(idx_hbm.at[pl.ds(cbase, ch)], idx_v)
            pltpu.async_copy(cb_hbm.at[idx_v], rows_v, sem).wait()
            pltpu.sync_copy(r_hbm.at[pl.ds(cbase, ch)], r_v)
            pltpu.sync_copy(x0_hbm.at[pl.ds(cbase, ch)], x_v)

            def tbody(t, acc):
                for j in range(d // L):
                    sl = pl.ds(j * L, L)
                    dlt = r_v[t, sl] - rows_v[t, sl]
                    x_v[t, sl] = x_v[t, sl] - dlt
                    acc = acc + dlt * dlt
                return acc
            acc_v[...] = lax.fori_loop(0, ch, tbody, acc_v[...])
            pltpu.sync_copy(x_v, qout_hbm.at[pl.ds(cbase, ch)])
        pltpu.sync_copy(acc_v, loss_hbm.at[wid])

    mesh = plsc.VectorSubcoreMesh(core_axis_name="c", subcore_axis_name="s")
    return pl.kernel(
        body,
        out_type=(jax.ShapeDtypeStruct((n, d), jnp.float32),
                  jax.ShapeDtypeStruct((NW, L), jnp.float32)),
        mesh=mesh,
        scratch_types=(pltpu.VMEM((ch,), jnp.int32),
                       pltpu.VMEM((ch, d), jnp.float32),
                       pltpu.VMEM((ch, d), jnp.float32),
                       pltpu.VMEM((ch, d), jnp.float32),
                       pltpu.VMEM((L,), jnp.float32),
                       pltpu.SemaphoreType.DMA))


@functools.lru_cache(maxsize=None)
def _make_sc_hist(n, k):
    perw = n // NW

    def body(idx_hbm, counts_hbm, idx_v, counts_v):
        wid = lax.axis_index("s") * NC + lax.axis_index("c")
        base = wid * perw
        zeros = jnp.zeros((L,), jnp.float32)
        ones = jnp.ones((L,), jnp.float32)
        lane = lax.iota(jnp.int32, L)
        one0 = jnp.where(lane == 0, ones, zeros)

        def zbody(i, c):
            counts_v[pl.ds(i * L, L)] = zeros
            return c
        lax.fori_loop(0, k // L + 1, zbody, 0)
        pltpu.sync_copy(idx_hbm.at[pl.ds(base, perw)], idx_v)
        for v in range(perw // L):
            iv = idx_v[pl.ds(v * L, L)]
            for j in range(L):
                si = iv[j]
                cv = counts_v[pl.ds(si, L)]
                counts_v[pl.ds(si, L)] = cv + one0
        pltpu.sync_copy(counts_v.at[pl.ds(0, k)], counts_hbm.at[wid])

    mesh = plsc.VectorSubcoreMesh(core_axis_name="c", subcore_axis_name="s")
    return pl.kernel(
        body,
        out_type=jax.ShapeDtypeStruct((NW, k), jnp.float32),
        mesh=mesh,
        scratch_types=(pltpu.VMEM((perw,), jnp.int32),
                       pltpu.VMEM((k + L,), jnp.float32)))


def kernel(x, codebooks):
    b, d, t = x.shape
    nq, k, _ = codebooks.shape
    n = b * t
    cn = jnp.sum(codebooks ** 2, axis=-1)
    flat0 = jnp.transpose(x, (0, 2, 1)).reshape(n, d)

    tm = 2304 if n % 2304 == 0 else n
    tk = 2048 if k % 2048 == 0 else k

    flat = flat0
    xd = None
    idxs, s_list, counts_list = [], [], []
    losspart = None
    qout_flat = None
    for q in range(nq):
        outs = _argmin_call(flat, xd, codebooks[q], cn[q], tm, tk)
        if q == 0:
            (idx,) = outs
        else:
            idx, flat, s = outs
            s_list.append(s)
        idx1 = idx[:, 0]
        idxs.append(idx1.reshape(b, t))
        counts_list.append(_make_sc_hist(n, k)(idx1))
        if q < nq - 1:
            xd = _make_sc_gather(n, d, k)(codebooks[q], idx1)
        else:
            qout_flat, losspart = _make_sc_final(n, d, k)(
                codebooks[q], idx1, flat, flat0)

    losses = [jnp.sum(s) / (n * d) for s in s_list]
    losses.append(jnp.sum(losspart) / (n * d))
    perps = []
    for counts in counts_list:
        prob = jnp.sum(counts, axis=0) / n
        perps.append(jnp.exp(-jnp.sum(prob * jnp.log(prob + 1e-7))))

    all_indices = jnp.stack(idxs, axis=-1)
    vq_loss = sum(losses) / nq
    perplexity = sum(perps) / nq
    quantized_out = jnp.transpose(qout_flat.reshape(b, t, d), (0, 2, 1))
    return quantized_out, all_indices, vq_loss, perplexity
